# baseline (speedup 1.0000x reference)
import functools

import jax
import jax.numpy as jnp
from jax import lax
from jax.experimental import pallas as pl
from jax.experimental.pallas import tpu as pltpu
from jax.experimental.pallas import tpu_sc as plsc

NC = 2
NS = 16
NW = NC * NS


VCHUNK = 128
RBUF = 2


def _relayout_kernel(V, D, n_cols, table_t_hbm, tail_hbm, out_hbm,
                     in_v0, in_v1, out_v0, out_v1, tail_v,
                     isem0, isem1, osem0, osem1):
    wid = lax.axis_index("s") * NC + lax.axis_index("c")
    n_base = n_cols // NW
    n_extra = n_cols - n_base * NW
    extra = jnp.minimum(wid, n_extra)
    start = wid * n_base + extra
    n_mine = n_base + jnp.where(wid < n_extra, 1, 0)

    in_bufs = (in_v0, in_v1)
    out_bufs = (out_v0, out_v1)
    isems = (isem0, isem1)
    osems = (osem0, osem1)

    iota = lax.iota(jnp.int32, 16)
    dperm = [(iota + s) % 16 for s in range(16)]
    sd_off = [iota * D + (iota + s) % 16 for s in range(16)]

    def fire_in(i, b):
        v0 = (start + i) * VCHUNK
        pltpu.async_copy(table_t_hbm.at[:, pl.ds(v0, VCHUNK)],
                         in_bufs[b], isems[b])

    def wait_in(b):
        pltpu.make_async_copy(table_t_hbm.at[:, pl.ds(0, VCHUNK)],
                              in_bufs[b], isems[b]).wait()

    def transpose(b):
        in_b = in_bufs[b]
        out_b = out_bufs[b]

        @plsc.parallel_loop(0, VCHUNK // 16, unroll=2)
        def vblk(j):
            vvec = iota + j * 16
            for d0 in range(D // 16):
                for s in range(16):
                    dvec = dperm[s] + d0 * 16
                    vals = plsc.load_gather(in_b, [dvec, vvec])
                    offs = sd_off[s] + (j * 16 * D + d0 * 16)
                    plsc.store_scatter(out_b, [offs], vals)

    def fire_out(i, b):
        v0 = (start + i) * VCHUNK
        pltpu.async_copy(out_bufs[b], out_hbm.at[pl.ds(v0 * D, VCHUNK * D)],
                         osems[b])

    def wait_out(b):
        pltpu.make_async_copy(out_bufs[b],
                              out_hbm.at[pl.ds(0, VCHUNK * D)],
                              osems[b]).wait()

    fire_in(0, 0)

    def step(q, carry):
        for b in range(RBUF):
            i = q * RBUF + b

            @pl.when(i < n_mine)
            def _():
                @pl.when(i + 1 < n_mine)
                def _():
                    fire_in(i + 1, (b + 1) % RBUF)

                wait_in(b)

                @pl.when(i >= RBUF)
                def _():
                    wait_out(b)

                transpose(b)
                fire_out(i, b)
        return carry

    lax.fori_loop(0, (n_mine + RBUF - 1) // RBUF, step, 0)
    for b in range(RBUF):
        @pl.when(n_mine > b)
        def _():
            wait_out(b)

    n_tail = V * D - n_cols * VCHUNK * D

    @pl.when(wid == 0)
    def _():
        pltpu.sync_copy(tail_hbm, tail_v)
        pltpu.sync_copy(tail_v, out_hbm.at[pl.ds(n_cols * VCHUNK * D, n_tail)])


def _make_relayout(V, D):
    n_cols = V // VCHUNK
    n_tail = V * D - n_cols * VCHUNK * D
    mesh = plsc.VectorSubcoreMesh(core_axis_name="c", subcore_axis_name="s")
    return pl.kernel(
        functools.partial(_relayout_kernel, V, D, n_cols),
        out_type=jax.ShapeDtypeStruct((V * D,), jnp.float32),
        mesh=mesh,
        scratch_types=[
            pltpu.VMEM((D, VCHUNK), jnp.float32),
            pltpu.VMEM((D, VCHUNK), jnp.float32),
            pltpu.VMEM((VCHUNK * D,), jnp.float32),
            pltpu.VMEM((VCHUNK * D,), jnp.float32),
            pltpu.VMEM((n_tail,), jnp.float32),
            pltpu.SemaphoreType.DMA,
            pltpu.SemaphoreType.DMA,
            pltpu.SemaphoreType.DMA,
            pltpu.SemaphoreType.DMA,
        ],
        compiler_params=pltpu.CompilerParams(use_tc_tiling_on_sc=True,
                                            needs_layout_passes=False),
    )


BBLK = 128
GRING = 4


def _gather_kernel(H, D, table_hbm, idx_hbm, out_hbm,
                   idx_v, gb0, gb1, gb2, gb3, tb0, tb1, tb2, tb3,
                   gsem0, gsem1, gsem2, gsem3, osem0, osem1, osem2, osem3):
    wid = lax.axis_index("s") * NC + lax.axis_index("c")
    gbufs = (gb0, gb1, gb2, gb3)
    tbufs = (tb0, tb1, tb2, tb3)
    gsems = (gsem0, gsem1, gsem2, gsem3)
    osems = (osem0, osem1, osem2, osem3)

    pltpu.sync_copy(idx_hbm.at[:, pl.ds(wid * BBLK, BBLK)], idx_v)

    iota = lax.iota(jnp.int32, 16)
    dperm = [(iota + s) % 16 for s in range(16)]

    def fire_gather(h, p):
        pltpu.async_copy(table_hbm.at[idx_v.at[h]], gbufs[p], gsems[p])

    def wait_gather(p):
        pltpu.make_async_copy(table_hbm.at[idx_v.at[0]], gbufs[p],
                              gsems[p]).wait()

    def transpose(p):
        gb = gbufs[p]
        tb = tbufs[p]

        @plsc.parallel_loop(0, BBLK // 16, unroll=2)
        def bblk_loop(j):
            bvec = iota + j * 16
            for d0 in range(D // 16):
                for s in range(16):
                    dvec = dperm[s] + d0 * 16
                    vals = plsc.load_gather(gb, [bvec, dvec])
                    i0 = dvec >> 3
                    i1 = dvec & 7
                    plsc.store_scatter(tb, [i0, i1, bvec], vals)

    def fire_write(h, p):
        pltpu.async_copy(tbufs[p], out_hbm.at[h, :, wid], osems[p])

    def wait_write(p):
        pltpu.make_async_copy(tbufs[p], out_hbm.at[0, :, wid],
                              osems[p]).wait()

    for p in range(GRING):
        fire_gather(p, p)

    def step(q, carry):
        for p in range(GRING):
            h = q * GRING + p
            wait_gather(p)

            @pl.when(h >= GRING)
            def _():
                wait_write(p)

            transpose(p)
            fire_write(h, p)

            @pl.when(h + GRING < H)
            def _():
                fire_gather(h + GRING, p)
        return carry

    lax.fori_loop(0, H // GRING, step, 0)
    for p in range(GRING):
        wait_write(p)


def _make_gather(B, H, V, D):
    mesh = plsc.VectorSubcoreMesh(core_axis_name="c", subcore_axis_name="s")
    return pl.kernel(
        functools.partial(_gather_kernel, H, D),
        out_type=jax.ShapeDtypeStruct((H, D // 8, B // BBLK, 8, BBLK),
                                      jnp.float32),
        mesh=mesh,
        scratch_types=[
            pltpu.VMEM((H, BBLK), jnp.int32),
            pltpu.VMEM((BBLK, D), jnp.float32),
            pltpu.VMEM((BBLK, D), jnp.float32),
            pltpu.VMEM((BBLK, D), jnp.float32),
            pltpu.VMEM((BBLK, D), jnp.float32),
            pltpu.VMEM((D // 8, 8, BBLK), jnp.float32),
            pltpu.VMEM((D // 8, 8, BBLK), jnp.float32),
            pltpu.VMEM((D // 8, 8, BBLK), jnp.float32),
            pltpu.VMEM((D // 8, 8, BBLK), jnp.float32),
            pltpu.SemaphoreType.DMA,
            pltpu.SemaphoreType.DMA,
            pltpu.SemaphoreType.DMA,
            pltpu.SemaphoreType.DMA,
            pltpu.SemaphoreType.DMA,
            pltpu.SemaphoreType.DMA,
            pltpu.SemaphoreType.DMA,
            pltpu.SemaphoreType.DMA,
        ],
        compiler_params=pltpu.CompilerParams(use_tc_tiling_on_sc=False,
                                            needs_layout_passes=False),
    )


def kernel(input_variable, weight):
    B, H = input_variable.shape
    V, D = weight.shape
    idxT = input_variable.T.astype(jnp.int32)
    n_cols = V // VCHUNK
    tail = weight[n_cols * VCHUNK:].reshape(-1)
    flat = _make_relayout(V, D)(weight.T, tail)
    table = flat.reshape(V, D)
    out5 = _make_gather(B, H, V, D)(table, idxT)
    return out5.transpose(0, 1, 3, 2, 4).reshape(H, D, B).transpose(2, 0, 1)


# speedup vs baseline: 1.3454x; 1.3454x over previous
import functools

import jax
import jax.numpy as jnp
from jax import lax
from jax.experimental import pallas as pl
from jax.experimental.pallas import tpu as pltpu
from jax.experimental.pallas import tpu_sc as plsc

NC = 2
NS = 16
NW = NC * NS


VCHUNK = 128
RBUF = 2


def _relayout_kernel(V, D, n_cols, table_t_hbm, tail_hbm, out_hbm,
                     in_v0, in_v1, out_v0, out_v1, tail_v,
                     isem0, isem1, osem0, osem1):
    wid = lax.axis_index("s") * NC + lax.axis_index("c")
    n_base = n_cols // NW
    n_extra = n_cols - n_base * NW
    extra = jnp.minimum(wid, n_extra)
    start = wid * n_base + extra
    n_mine = n_base + jnp.where(wid < n_extra, 1, 0)

    in_bufs = (in_v0, in_v1)
    out_bufs = (out_v0, out_v1)
    isems = (isem0, isem1)
    osems = (osem0, osem1)

    iota = lax.iota(jnp.int32, 16)
    zero = iota * 0
    ld_off = [((iota + s) % 16) * VCHUNK + iota for s in range(16)]
    st_off = [iota * D + (iota + s) % 16 for s in range(16)]

    def fire_in(i, b):
        v0 = (start + i) * VCHUNK
        pltpu.async_copy(table_t_hbm.at[:, pl.ds(v0, VCHUNK)],
                         in_bufs[b], isems[b])

    def wait_in(b):
        pltpu.make_async_copy(table_t_hbm.at[:, pl.ds(0, VCHUNK)],
                              in_bufs[b], isems[b]).wait()

    def transpose(b):
        in_b = in_bufs[b]
        out_b = out_bufs[b]

        def vblk(j, carry):
            for d0 in range(D // 16):
                for s in range(16):
                    lo = ld_off[s] + (d0 * 16 * VCHUNK + j * 16)
                    vals = plsc.load_gather(in_b, [zero, lo])
                    so = st_off[s] + (j * 16 * D + d0 * 16)
                    plsc.store_scatter(out_b, [so], vals)
            return carry

        lax.fori_loop(0, VCHUNK // 16, vblk, 0)

    def fire_out(i, b):
        v0 = (start + i) * VCHUNK
        pltpu.async_copy(out_bufs[b], out_hbm.at[pl.ds(v0 * D, VCHUNK * D)],
                         osems[b])

    def wait_out(b):
        pltpu.make_async_copy(out_bufs[b],
                              out_hbm.at[pl.ds(0, VCHUNK * D)],
                              osems[b]).wait()

    fire_in(0, 0)

    def step(q, carry):
        for b in range(RBUF):
            i = q * RBUF + b

            @pl.when(i < n_mine)
            def _():
                @pl.when(i + 1 < n_mine)
                def _():
                    fire_in(i + 1, (b + 1) % RBUF)

                wait_in(b)

                @pl.when(i >= RBUF)
                def _():
                    wait_out(b)

                transpose(b)
                fire_out(i, b)
        return carry

    lax.fori_loop(0, (n_mine + RBUF - 1) // RBUF, step, 0)
    for b in range(RBUF):
        @pl.when(n_mine > b)
        def _():
            wait_out(b)

    n_tail = V * D - n_cols * VCHUNK * D

    @pl.when(wid == 0)
    def _():
        pltpu.sync_copy(tail_hbm, tail_v)
        pltpu.sync_copy(tail_v, out_hbm.at[pl.ds(n_cols * VCHUNK * D, n_tail)])


def _make_relayout(V, D):
    n_cols = V // VCHUNK
    n_tail = V * D - n_cols * VCHUNK * D
    mesh = plsc.VectorSubcoreMesh(core_axis_name="c", subcore_axis_name="s")
    return pl.kernel(
        functools.partial(_relayout_kernel, V, D, n_cols),
        out_type=jax.ShapeDtypeStruct((V * D,), jnp.float32),
        mesh=mesh,
        scratch_types=[
            pltpu.VMEM((D, VCHUNK), jnp.float32),
            pltpu.VMEM((D, VCHUNK), jnp.float32),
            pltpu.VMEM((VCHUNK * D,), jnp.float32),
            pltpu.VMEM((VCHUNK * D,), jnp.float32),
            pltpu.VMEM((n_tail,), jnp.float32),
            pltpu.SemaphoreType.DMA,
            pltpu.SemaphoreType.DMA,
            pltpu.SemaphoreType.DMA,
            pltpu.SemaphoreType.DMA,
        ],
        compiler_params=pltpu.CompilerParams(use_tc_tiling_on_sc=True,
                                            needs_layout_passes=False),
    )


BBLK = 128
GRING = 4


def _gather_kernel(H, D, table_hbm, idx_hbm, out_hbm,
                   idx_v, gb0, gb1, gb2, gb3, tb0, tb1, tb2, tb3,
                   gsem0, gsem1, gsem2, gsem3, osem0, osem1, osem2, osem3):
    wid = lax.axis_index("s") * NC + lax.axis_index("c")
    gbufs = (gb0, gb1, gb2, gb3)
    tbufs = (tb0, tb1, tb2, tb3)
    gsems = (gsem0, gsem1, gsem2, gsem3)
    osems = (osem0, osem1, osem2, osem3)

    pltpu.sync_copy(idx_hbm.at[:, pl.ds(wid * BBLK, BBLK)], idx_v)

    iota = lax.iota(jnp.int32, 16)
    zero = iota * 0
    gl_off = [iota * D + (iota + s) % 16 for s in range(16)]
    gs_off = [((iota + s) % 16) * BBLK + iota for s in range(16)]

    def fire_gather(h, p):
        pltpu.async_copy(table_hbm.at[idx_v.at[h]], gbufs[p], gsems[p])

    def wait_gather(p):
        pltpu.make_async_copy(table_hbm.at[idx_v.at[0]], gbufs[p],
                              gsems[p]).wait()

    def transpose(p):
        gb = gbufs[p]
        tb = tbufs[p]

        def bblk_loop(j, carry):
            for d0 in range(D // 16):
                for s in range(16):
                    lo = gl_off[s] + (j * 16 * D + d0 * 16)
                    vals = plsc.load_gather(gb, [zero, lo])
                    so = gs_off[s] + (d0 * 16 * BBLK + j * 16)
                    plsc.store_scatter(tb, [zero, zero, so], vals)
            return carry

        lax.fori_loop(0, BBLK // 16, bblk_loop, 0)

    def fire_write(h, p):
        pltpu.async_copy(tbufs[p], out_hbm.at[h, :, wid], osems[p])

    def wait_write(p):
        pltpu.make_async_copy(tbufs[p], out_hbm.at[0, :, wid],
                              osems[p]).wait()

    for p in range(GRING):
        fire_gather(p, p)

    def step(q, carry):
        for p in range(GRING):
            h = q * GRING + p
            wait_gather(p)

            @pl.when(h >= GRING)
            def _():
                wait_write(p)

            transpose(p)
            fire_write(h, p)

            @pl.when(h + GRING < H)
            def _():
                fire_gather(h + GRING, p)
        return carry

    lax.fori_loop(0, H // GRING, step, 0)
    for p in range(GRING):
        wait_write(p)


def _make_gather(B, H, V, D):
    mesh = plsc.VectorSubcoreMesh(core_axis_name="c", subcore_axis_name="s")
    return pl.kernel(
        functools.partial(_gather_kernel, H, D),
        out_type=jax.ShapeDtypeStruct((H, D // 8, B // BBLK, 8, BBLK),
                                      jnp.float32),
        mesh=mesh,
        scratch_types=[
            pltpu.VMEM((H, BBLK), jnp.int32),
            pltpu.VMEM((BBLK, D), jnp.float32),
            pltpu.VMEM((BBLK, D), jnp.float32),
            pltpu.VMEM((BBLK, D), jnp.float32),
            pltpu.VMEM((BBLK, D), jnp.float32),
            pltpu.VMEM((D // 8, 8, BBLK), jnp.float32),
            pltpu.VMEM((D // 8, 8, BBLK), jnp.float32),
            pltpu.VMEM((D // 8, 8, BBLK), jnp.float32),
            pltpu.VMEM((D // 8, 8, BBLK), jnp.float32),
            pltpu.SemaphoreType.DMA,
            pltpu.SemaphoreType.DMA,
            pltpu.SemaphoreType.DMA,
            pltpu.SemaphoreType.DMA,
            pltpu.SemaphoreType.DMA,
            pltpu.SemaphoreType.DMA,
            pltpu.SemaphoreType.DMA,
            pltpu.SemaphoreType.DMA,
        ],
        compiler_params=pltpu.CompilerParams(use_tc_tiling_on_sc=False,
                                            needs_layout_passes=False),
    )


def kernel(input_variable, weight):
    B, H = input_variable.shape
    V, D = weight.shape
    idxT = input_variable.T.astype(jnp.int32)
    n_cols = V // VCHUNK
    tail = weight[n_cols * VCHUNK:].reshape(-1)
    flat = _make_relayout(V, D)(weight.T, tail)
    table = flat.reshape(V, D)
    out5 = _make_gather(B, H, V, D)(table, idxT)
    return out5.transpose(0, 1, 3, 2, 4).reshape(H, D, B).transpose(2, 0, 1)


# speedup vs baseline: 3.1621x; 2.3503x over previous
import functools

import jax
import jax.numpy as jnp
from jax import lax
from jax.experimental import pallas as pl
from jax.experimental.pallas import tpu as pltpu
from jax.experimental.pallas import tpu_sc as plsc

NC = 2
NS = 16
NW = NC * NS


VCHUNK = 128
RBUF = 2


def _relayout_kernel(V, D, n_cols, table_t_hbm, tail_hbm, out_hbm,
                     in_v0, in_v1, out_v0, out_v1, tail_v,
                     isem0, isem1, osem0, osem1):
    wid = lax.axis_index("s") * NC + lax.axis_index("c")
    n_base = n_cols // NW
    n_extra = n_cols - n_base * NW
    extra = jnp.minimum(wid, n_extra)
    start = wid * n_base + extra
    n_mine = n_base + jnp.where(wid < n_extra, 1, 0)

    in_bufs = (in_v0, in_v1)
    out_bufs = (out_v0, out_v1)
    isems = (isem0, isem1)
    osems = (osem0, osem1)

    iota = lax.iota(jnp.int32, 16)
    zero = iota * 0
    ld_off = [((iota + s) % 16) * VCHUNK + iota for s in range(16)]
    st_off = [iota * D + (iota + s) % 16 for s in range(16)]

    def fire_in(i, b):
        v0 = (start + i) * VCHUNK
        pltpu.async_copy(table_t_hbm.at[:, pl.ds(v0, VCHUNK)],
                         in_bufs[b], isems[b])

    def wait_in(b):
        pltpu.make_async_copy(table_t_hbm.at[:, pl.ds(0, VCHUNK)],
                              in_bufs[b], isems[b]).wait()

    def transpose(b):
        in_b = in_bufs[b]
        out_b = out_bufs[b]

        def vblk(j, carry):
            for d0 in range(D // 16):
                vals = [plsc.load_gather(
                    in_b, [zero, ld_off[s] + (d0 * 16 * VCHUNK + j * 16)])
                    for s in range(16)]
                for s in range(16):
                    so = st_off[s] + (j * 16 * D + d0 * 16)
                    plsc.store_scatter(out_b, [so], vals[s])
            return carry

        lax.fori_loop(0, VCHUNK // 16, vblk, 0)

    def fire_out(i, b):
        v0 = (start + i) * VCHUNK
        pltpu.async_copy(out_bufs[b], out_hbm.at[pl.ds(v0 * D, VCHUNK * D)],
                         osems[b])

    def wait_out(b):
        pltpu.make_async_copy(out_bufs[b],
                              out_hbm.at[pl.ds(0, VCHUNK * D)],
                              osems[b]).wait()

    fire_in(0, 0)

    def step(q, carry):
        for b in range(RBUF):
            i = q * RBUF + b

            @pl.when(i < n_mine)
            def _():
                @pl.when(i + 1 < n_mine)
                def _():
                    fire_in(i + 1, (b + 1) % RBUF)

                wait_in(b)

                @pl.when(i >= RBUF)
                def _():
                    wait_out(b)

                transpose(b)
                fire_out(i, b)
        return carry

    lax.fori_loop(0, (n_mine + RBUF - 1) // RBUF, step, 0)
    for b in range(RBUF):
        @pl.when(n_mine > b)
        def _():
            wait_out(b)

    n_tail = V * D - n_cols * VCHUNK * D

    @pl.when(wid == 0)
    def _():
        pltpu.sync_copy(tail_hbm, tail_v)
        pltpu.sync_copy(tail_v, out_hbm.at[pl.ds(n_cols * VCHUNK * D, n_tail)])


def _make_relayout(V, D):
    n_cols = V // VCHUNK
    n_tail = V * D - n_cols * VCHUNK * D
    mesh = plsc.VectorSubcoreMesh(core_axis_name="c", subcore_axis_name="s")
    return pl.kernel(
        functools.partial(_relayout_kernel, V, D, n_cols),
        out_type=jax.ShapeDtypeStruct((V * D,), jnp.float32),
        mesh=mesh,
        scratch_types=[
            pltpu.VMEM((D, VCHUNK), jnp.float32),
            pltpu.VMEM((D, VCHUNK), jnp.float32),
            pltpu.VMEM((VCHUNK * D,), jnp.float32),
            pltpu.VMEM((VCHUNK * D,), jnp.float32),
            pltpu.VMEM((n_tail,), jnp.float32),
            pltpu.SemaphoreType.DMA,
            pltpu.SemaphoreType.DMA,
            pltpu.SemaphoreType.DMA,
            pltpu.SemaphoreType.DMA,
        ],
        compiler_params=pltpu.CompilerParams(use_tc_tiling_on_sc=True,
                                            needs_layout_passes=False),
    )


BBLK = 128
GRING = 4


def _gather_kernel(H, D, table_hbm, idx_hbm, out_hbm,
                   idx_v, gb0, gb1, gb2, gb3, tb0, tb1, tb2, tb3,
                   gsem0, gsem1, gsem2, gsem3, osem0, osem1, osem2, osem3):
    wid = lax.axis_index("s") * NC + lax.axis_index("c")
    gbufs = (gb0, gb1, gb2, gb3)
    tbufs = (tb0, tb1, tb2, tb3)
    gsems = (gsem0, gsem1, gsem2, gsem3)
    osems = (osem0, osem1, osem2, osem3)

    pltpu.sync_copy(idx_hbm.at[:, pl.ds(wid * BBLK, BBLK)], idx_v)

    iota = lax.iota(jnp.int32, 16)
    zero = iota * 0
    gl_off = [iota * D + (iota + s) % 16 for s in range(16)]
    gs_off = [((iota + s) % 16) * BBLK + iota for s in range(16)]

    def fire_gather(h, p):
        pltpu.async_copy(table_hbm.at[idx_v.at[h]], gbufs[p], gsems[p])

    def wait_gather(p):
        pltpu.make_async_copy(table_hbm.at[idx_v.at[0]], gbufs[p],
                              gsems[p]).wait()

    def transpose(p):
        gb = gbufs[p]
        tb = tbufs[p]

        def bblk_loop(j, carry):
            for d0 in range(D // 16):
                vals = [plsc.load_gather(
                    gb, [zero, gl_off[s] + (j * 16 * D + d0 * 16)])
                    for s in range(16)]
                for s in range(16):
                    so = gs_off[s] + (d0 * 16 * BBLK + j * 16)
                    plsc.store_scatter(tb, [zero, zero, so], vals[s])
            return carry

        lax.fori_loop(0, BBLK // 16, bblk_loop, 0)

    def fire_write(h, p):
        pltpu.async_copy(tbufs[p], out_hbm.at[h, :, wid], osems[p])

    def wait_write(p):
        pltpu.make_async_copy(tbufs[p], out_hbm.at[0, :, wid],
                              osems[p]).wait()

    for p in range(GRING):
        fire_gather(p, p)

    def step(q, carry):
        for p in range(GRING):
            h = q * GRING + p
            wait_gather(p)

            @pl.when(h >= GRING)
            def _():
                wait_write(p)

            transpose(p)
            fire_write(h, p)

            @pl.when(h + GRING < H)
            def _():
                fire_gather(h + GRING, p)
        return carry

    lax.fori_loop(0, H // GRING, step, 0)
    for p in range(GRING):
        wait_write(p)


def _make_gather(B, H, V, D):
    mesh = plsc.VectorSubcoreMesh(core_axis_name="c", subcore_axis_name="s")
    return pl.kernel(
        functools.partial(_gather_kernel, H, D),
        out_type=jax.ShapeDtypeStruct((H, D // 8, B // BBLK, 8, BBLK),
                                      jnp.float32),
        mesh=mesh,
        scratch_types=[
            pltpu.VMEM((H, BBLK), jnp.int32),
            pltpu.VMEM((BBLK, D), jnp.float32),
            pltpu.VMEM((BBLK, D), jnp.float32),
            pltpu.VMEM((BBLK, D), jnp.float32),
            pltpu.VMEM((BBLK, D), jnp.float32),
            pltpu.VMEM((D // 8, 8, BBLK), jnp.float32),
            pltpu.VMEM((D // 8, 8, BBLK), jnp.float32),
            pltpu.VMEM((D // 8, 8, BBLK), jnp.float32),
            pltpu.VMEM((D // 8, 8, BBLK), jnp.float32),
            pltpu.SemaphoreType.DMA,
            pltpu.SemaphoreType.DMA,
            pltpu.SemaphoreType.DMA,
            pltpu.SemaphoreType.DMA,
            pltpu.SemaphoreType.DMA,
            pltpu.SemaphoreType.DMA,
            pltpu.SemaphoreType.DMA,
            pltpu.SemaphoreType.DMA,
        ],
        compiler_params=pltpu.CompilerParams(use_tc_tiling_on_sc=False,
                                            needs_layout_passes=False),
    )


def kernel(input_variable, weight):
    B, H = input_variable.shape
    V, D = weight.shape
    idxT = input_variable.T.astype(jnp.int32)
    n_cols = V // VCHUNK
    tail = weight[n_cols * VCHUNK:].reshape(-1)
    flat = _make_relayout(V, D)(weight.T, tail)
    table = flat.reshape(V, D)
    out5 = _make_gather(B, H, V, D)(table, idxT)
    return out5.transpose(0, 1, 3, 2, 4).reshape(H, D, B).transpose(2, 0, 1)
